# Initial kernel scaffold; baseline (speedup 1.0000x reference)
import jax
import jax.numpy as jnp
from jax.experimental import pallas as pl


def kernel(feature, adj_values, W, Wq, bq, Wk, bk, Wv, bv, edge_index):
    raise NotImplementedError("write your pallas kernel here")


# speedup vs baseline: 2.2503x; 2.2503x over previous
import functools

import jax
import jax.numpy as jnp
from jax import lax
from jax.experimental import pallas as pl
from jax.experimental.pallas import tpu as pltpu
from jax.experimental.pallas import tpu_sc as plsc

NC = 2
NS = 16
NW = NC * NS
CHUNK = 128
LANES = 16


def _mm_relu_body(f_ref, w_ref, o_ref):
    o_ref[...] = jnp.maximum(
        jnp.dot(f_ref[...], w_ref[...], preferred_element_type=jnp.float32), 0.0
    )


def _merge_body(p_ref, o_ref):
    n = o_ref.shape[0]
    o_ref[...] = p_ref[0, :n, :] + p_ref[1, :n, :]


def _final_body(a1_ref, q_ref, wv_ref, bv_ref, o_ref):
    n = o_ref.shape[0]
    s = a1_ref[...] + q_ref[0, :n, :] + q_ref[1, :n, :]
    o_ref[...] = (
        lax.dot_general(
            s, wv_ref[...], (((1,), (1,)), ((), ())),
            preferred_element_type=jnp.float32,
        )
        + 2.0 * bv_ref[...]
    )


def _make_hop(n_pad, d, cpw):
    rows_per_tile = n_pad // NS

    def hop_body(x_hbm, row_hbm, col_hbm, valx_hbm, zrow_hbm, out_hbm,
                 row_v, col_v, valx_v, rows_v, acc_sh, sem):
        c = lax.axis_index("c")
        s = lax.axis_index("s")
        wid = c * NS + s

        pltpu.sync_copy(row_hbm.at[pl.ds(wid * cpw, cpw)], row_v)
        pltpu.sync_copy(col_hbm.at[pl.ds(wid * cpw, cpw)], col_v)

        pltpu.sync_copy(zrow_hbm, rows_v)
        for k in range(rows_per_tile // CHUNK):
            pltpu.sync_copy(
                rows_v,
                acc_sh.at[pl.ds(s * rows_per_tile + k * CHUNK, CHUNK)],
            )
        plsc.subcore_barrier()

        def chunk_body(j, carry):
            pltpu.sync_copy(
                valx_hbm.at[pl.ds((wid * cpw + j) * CHUNK * LANES, CHUNK * LANES)],
                valx_v,
            )
            pltpu.async_copy(x_hbm.at[col_v.at[j]], rows_v, sem).wait()

            def edge_body(e, carry2):
                v16 = valx_v[pl.ds(e * LANES, LANES)]
                for qq in range(d // LANES):
                    sl = pl.ds(qq * LANES, LANES)
                    rows_v[e, sl] = rows_v[e, sl] * v16
                return carry2

            lax.fori_loop(0, CHUNK, edge_body, 0)
            pltpu.sync_copy(rows_v, acc_sh.at[row_v.at[j]], add=True)
            return carry

        lax.fori_loop(0, cpw, chunk_body, 0)

        plsc.subcore_barrier()
        pltpu.sync_copy(
            acc_sh.at[pl.ds(s * rows_per_tile, rows_per_tile)],
            out_hbm.at[c].at[pl.ds(s * rows_per_tile, rows_per_tile)],
        )

    mesh = plsc.VectorSubcoreMesh(core_axis_name="c", subcore_axis_name="s")
    return pl.kernel(
        hop_body,
        out_type=jax.ShapeDtypeStruct((NC, n_pad, d), jnp.float32),
        mesh=mesh,
        scratch_types=[
            pltpu.VMEM((cpw, CHUNK), jnp.int32),
            pltpu.VMEM((cpw, CHUNK), jnp.int32),
            pltpu.VMEM((CHUNK * LANES,), jnp.float32),
            pltpu.VMEM((CHUNK, d), jnp.float32),
            pltpu.VMEM_SHARED((n_pad, d), jnp.float32),
            pltpu.SemaphoreType.DMA,
        ],
    )


@jax.jit
def kernel(feature, adj_values, W, Wq, bq, Wk, bk, Wv, bv, edge_index):
    n, f = feature.shape
    d = W.shape[1]
    e = adj_values.shape[0]

    cpw = -(-e // (NW * CHUNK))
    cpw = -(-cpw // 8) * 8
    e_pad = NW * CHUNK * cpw
    pad = e_pad - e
    row = jnp.pad(edge_index[0], (0, pad)).reshape(-1, CHUNK)
    col = jnp.pad(edge_index[1], (0, pad)).reshape(-1, CHUNK)
    val = jnp.pad(adj_values, (0, pad))
    valx = jnp.broadcast_to(val[:, None], (e_pad, LANES)).reshape(-1)
    zrow = jnp.zeros((CHUNK, d), jnp.float32)
    n_pad = -(-n // (NS * CHUNK)) * NS * CHUNK

    support = pl.pallas_call(
        _mm_relu_body,
        out_shape=jax.ShapeDtypeStruct((n, d), jnp.float32),
    )(feature, W)

    hop = _make_hop(n_pad, d, cpw)
    p = hop(support, row, col, valx, zrow)
    agg1 = pl.pallas_call(
        _merge_body,
        out_shape=jax.ShapeDtypeStruct((n, d), jnp.float32),
    )(p)
    q = hop(agg1, row, col, valx, zrow)
    out = pl.pallas_call(
        _final_body,
        out_shape=jax.ShapeDtypeStruct((n, d), jnp.float32),
    )(agg1, q, Wv, bv.reshape(1, d))
    return out


# speedup vs baseline: 2.9215x; 1.2983x over previous
import jax
import jax.numpy as jnp
from jax import lax
from jax.experimental import pallas as pl
from jax.experimental.pallas import tpu as pltpu
from jax.experimental.pallas import tpu_sc as plsc

NC = 2
NS = 16
NW = NC * NS
CHUNK = 128
LANES = 16


def _mm_relu_body(f_ref, w_ref, o_ref):
    o_ref[...] = jnp.maximum(
        jnp.dot(f_ref[...], w_ref[...], preferred_element_type=jnp.float32), 0.0
    )


def _merge_body(p_ref, o_ref):
    n = o_ref.shape[0]
    o_ref[...] = p_ref[0, :n, :] + p_ref[1, :n, :]


def _final_body(a1_ref, q_ref, wv_ref, bv_ref, o_ref):
    n = o_ref.shape[0]
    s = a1_ref[...] + q_ref[0, :n, :] + q_ref[1, :n, :]
    o_ref[...] = (
        lax.dot_general(
            s, wv_ref[...], (((1,), (1,)), ((), ())),
            preferred_element_type=jnp.float32,
        )
        + 2.0 * bv_ref[...]
    )


def _make_hop(n_pad, d, cpw):
    rows_per_tile = n_pad // NS
    NB = 2
    NI = 3
    CL = CHUNK * LANES

    def hop_body(x_hbm, row_hbm, col_hbm, valx_hbm, zrow_hbm, out_hbm,
                 row_v, col_v, valx_v, rows_v, acc_sh, gsem, isem, ssem):
        c = lax.axis_index("c")
        s = lax.axis_index("s")
        wid = c * NS + s

        pltpu.sync_copy(zrow_hbm, rows_v.at[0])
        for k in range(rows_per_tile // CHUNK):
            pltpu.sync_copy(
                rows_v.at[0],
                acc_sh.at[pl.ds(s * rows_per_tile + k * CHUNK, CHUNK)],
            )
        plsc.subcore_barrier()

        def start_idx(j, b):
            pltpu.async_copy(row_hbm.at[wid * cpw + j], row_v.at[b], isem.at[b])
            pltpu.async_copy(col_hbm.at[wid * cpw + j], col_v.at[b], isem.at[b])
            pltpu.async_copy(
                valx_hbm.at[pl.ds((wid * cpw + j) * CL, CL)],
                valx_v.at[b], isem.at[b],
            )

        def wait_idx(b):
            pltpu.make_async_copy(row_hbm.at[0], row_v.at[b], isem.at[b]).wait()
            pltpu.make_async_copy(col_hbm.at[0], col_v.at[b], isem.at[b]).wait()
            pltpu.make_async_copy(
                valx_hbm.at[pl.ds(0, CL)], valx_v.at[b], isem.at[b]
            ).wait()

        start_idx(0, 0)
        wait_idx(0)
        pltpu.async_copy(x_hbm.at[col_v.at[0]], rows_v.at[0], gsem.at[0])
        start_idx(1, 1)

        def chunk_body(j, carry):
            b = lax.rem(j, NB)
            nb = lax.rem(j + 1, NB)
            bi = lax.rem(j, NI)
            nbi = lax.rem(j + 1, NI)

            @pl.when(j >= 1)
            def _():
                pltpu.make_async_copy(
                    rows_v.at[nb], acc_sh.at[row_v.at[0]], ssem.at[nb]
                ).wait()

            @pl.when(j + 1 < cpw)
            def _():
                wait_idx(nbi)
                pltpu.async_copy(
                    x_hbm.at[col_v.at[nbi]], rows_v.at[nb], gsem.at[nb]
                )

            @pl.when(j + 2 < cpw)
            def _():
                start_idx(j + 2, lax.rem(j + 2, NI))

            pltpu.make_async_copy(
                x_hbm.at[pl.ds(0, CHUNK)], rows_v.at[b], gsem.at[b]
            ).wait()

            @plsc.parallel_loop(0, CHUNK, unroll=4)
            def _(e2):
                v16 = valx_v[bi, pl.ds(e2 * LANES, LANES)]
                for qq in range(d // LANES):
                    sl = pl.ds(qq * LANES, LANES)
                    rows_v[b, e2, sl] = rows_v[b, e2, sl] * v16

            pltpu.async_copy(
                rows_v.at[b], acc_sh.at[row_v.at[bi]], ssem.at[b], add=True
            )
            return carry

        lax.fori_loop(0, cpw, chunk_body, 0)
        lb = lax.rem(jnp.int32(cpw - 1), NB)
        pltpu.make_async_copy(
            rows_v.at[lb], acc_sh.at[row_v.at[0]], ssem.at[lb]
        ).wait()

        plsc.subcore_barrier()
        pltpu.sync_copy(
            acc_sh.at[pl.ds(s * rows_per_tile, rows_per_tile)],
            out_hbm.at[c].at[pl.ds(s * rows_per_tile, rows_per_tile)],
        )

    mesh = plsc.VectorSubcoreMesh(core_axis_name="c", subcore_axis_name="s")
    return pl.kernel(
        hop_body,
        out_type=jax.ShapeDtypeStruct((NC, n_pad, d), jnp.float32),
        mesh=mesh,
        scratch_types=[
            pltpu.VMEM((NI, CHUNK), jnp.int32),
            pltpu.VMEM((NI, CHUNK), jnp.int32),
            pltpu.VMEM((NI, CL), jnp.float32),
            pltpu.VMEM((NB, CHUNK, d), jnp.float32),
            pltpu.VMEM_SHARED((n_pad, d), jnp.float32),
            pltpu.SemaphoreType.DMA((NB,)),
            pltpu.SemaphoreType.DMA((NI,)),
            pltpu.SemaphoreType.DMA((NB,)),
        ],
    )


@jax.jit
def kernel(feature, adj_values, W, Wq, bq, Wk, bk, Wv, bv, edge_index):
    n, f = feature.shape
    d = W.shape[1]
    e = adj_values.shape[0]

    cpw = -(-e // (NW * CHUNK))
    cpw = -(-cpw // 8) * 8
    e_pad = NW * CHUNK * cpw
    pad = e_pad - e
    row = jnp.pad(edge_index[0], (0, pad)).reshape(-1, CHUNK)
    col = jnp.pad(edge_index[1], (0, pad)).reshape(-1, CHUNK)
    val = jnp.pad(adj_values, (0, pad))
    valx = jnp.broadcast_to(val[:, None], (e_pad, LANES)).reshape(-1)
    zrow = jnp.zeros((CHUNK, d), jnp.float32)
    n_pad = -(-n // (NS * CHUNK)) * NS * CHUNK

    support = pl.pallas_call(
        _mm_relu_body,
        out_shape=jax.ShapeDtypeStruct((n, d), jnp.float32),
    )(feature, W)

    hop = _make_hop(n_pad, d, cpw)
    p = hop(support, row, col, valx, zrow)
    agg1 = pl.pallas_call(
        _merge_body,
        out_shape=jax.ShapeDtypeStruct((n, d), jnp.float32),
    )(p)
    q = hop(agg1, row, col, valx, zrow)
    out = pl.pallas_call(
        _final_body,
        out_shape=jax.ShapeDtypeStruct((n, d), jnp.float32),
    )(agg1, q, Wv, bv.reshape(1, d))
    return out


# speedup vs baseline: 6.2497x; 2.1392x over previous
import jax
import jax.numpy as jnp
from jax import lax
from jax.experimental import pallas as pl
from jax.experimental.pallas import tpu as pltpu
from jax.experimental.pallas import tpu_sc as plsc

NC = 2
NS = 16
NW = NC * NS
CHUNK = 128
LANES = 16


def _mm_relu_body(f_ref, w_ref, o_ref):
    o_ref[...] = jnp.maximum(
        jnp.dot(f_ref[...], w_ref[...], preferred_element_type=jnp.float32), 0.0
    )


def _merge_body(p_ref, o_ref):
    n = o_ref.shape[0]
    o_ref[...] = p_ref[0, :n, :] + p_ref[1, :n, :]


def _final_body(a1_ref, q_ref, wv_ref, bv_ref, o_ref):
    n = o_ref.shape[0]
    s = a1_ref[...] + q_ref[0, :n, :] + q_ref[1, :n, :]
    o_ref[...] = (
        lax.dot_general(
            s, wv_ref[...], (((1,), (1,)), ((), ())),
            preferred_element_type=jnp.float32,
        )
        + 2.0 * bv_ref[...]
    )


def _make_hop(n_pad, d, cpw):
    rows_per_tile = n_pad // NS
    NB = 2
    NI = 3
    CL = CHUNK * LANES

    def hop_body(x_hbm, row_hbm, col_hbm, valx_hbm, zrow_hbm, out_hbm,
                 row_v, col_v, valx_v, rows_v, acc_sh, gsem, isem, ssem):
        c = lax.axis_index("c")
        s = lax.axis_index("s")
        wid = c * NS + s

        pltpu.sync_copy(zrow_hbm, rows_v.at[0])
        for k in range(rows_per_tile // CHUNK):
            pltpu.sync_copy(
                rows_v.at[0],
                acc_sh.at[pl.ds(s * rows_per_tile + k * CHUNK, CHUNK)],
            )
        plsc.subcore_barrier()

        def start_idx(j, b):
            pltpu.async_copy(row_hbm.at[wid * cpw + j], row_v.at[b], isem.at[b])
            pltpu.async_copy(col_hbm.at[wid * cpw + j], col_v.at[b], isem.at[b])
            pltpu.async_copy(
                valx_hbm.at[pl.ds((wid * cpw + j) * CL, CL)],
                valx_v.at[b], isem.at[b],
            )

        def wait_idx(b):
            pltpu.make_async_copy(row_hbm.at[0], row_v.at[b], isem.at[b]).wait()
            pltpu.make_async_copy(col_hbm.at[0], col_v.at[b], isem.at[b]).wait()
            pltpu.make_async_copy(
                valx_hbm.at[pl.ds(0, CL)], valx_v.at[b], isem.at[b]
            ).wait()

        start_idx(0, 0)
        wait_idx(0)
        pltpu.async_copy(x_hbm.at[col_v.at[0]], rows_v.at[0], gsem.at[0])
        start_idx(1, 1)

        def chunk_body(j, carry):
            b = lax.rem(j, NB)
            nb = lax.rem(j + 1, NB)
            bi = lax.rem(j, NI)
            nbi = lax.rem(j + 1, NI)

            @pl.when(j >= 1)
            def _():
                pltpu.make_async_copy(
                    rows_v.at[nb], acc_sh.at[row_v.at[0]], ssem.at[nb]
                ).wait()

            @pl.when(j + 1 < cpw)
            def _():
                wait_idx(nbi)
                pltpu.async_copy(
                    x_hbm.at[col_v.at[nbi]], rows_v.at[nb], gsem.at[nb]
                )

            @pl.when(j + 2 < cpw)
            def _():
                start_idx(j + 2, lax.rem(j + 2, NI))

            pltpu.make_async_copy(
                x_hbm.at[pl.ds(0, CHUNK)], rows_v.at[b], gsem.at[b]
            ).wait()

            @plsc.parallel_loop(0, CHUNK, unroll=4)
            def _(e2):
                v16 = valx_v[bi, pl.ds(e2 * LANES, LANES)]
                for qq in range(d // LANES):
                    sl = pl.ds(qq * LANES, LANES)
                    rows_v[b, e2, sl] = rows_v[b, e2, sl] * v16

            pltpu.async_copy(
                rows_v.at[b], acc_sh.at[row_v.at[bi]], ssem.at[b], add=True
            )
            return carry

        lax.fori_loop(0, cpw, chunk_body, 0)
        lb = lax.rem(jnp.int32(cpw - 1), NB)
        pltpu.make_async_copy(
            rows_v.at[lb], acc_sh.at[row_v.at[0]], ssem.at[lb]
        ).wait()

        plsc.subcore_barrier()
        pltpu.sync_copy(
            acc_sh.at[pl.ds(s * rows_per_tile, rows_per_tile)],
            out_hbm.at[c].at[pl.ds(s * rows_per_tile, rows_per_tile)],
        )

    mesh = plsc.VectorSubcoreMesh(core_axis_name="c", subcore_axis_name="s")
    return pl.kernel(
        hop_body,
        out_type=jax.ShapeDtypeStruct((NC, n_pad, d), jnp.float32),
        mesh=mesh,
        scratch_types=[
            pltpu.VMEM((NI, CHUNK), jnp.int32),
            pltpu.VMEM((NI, CHUNK), jnp.int32),
            pltpu.VMEM((NI, CL), jnp.float32),
            pltpu.VMEM((NB, CHUNK, d), jnp.float32),
            pltpu.VMEM_SHARED((n_pad, d), jnp.float32),
            pltpu.SemaphoreType.DMA((NB,)),
            pltpu.SemaphoreType.DMA((NI,)),
            pltpu.SemaphoreType.DMA((NB,)),
        ],
    )


@jax.jit
def kernel(feature, adj_values, W, Wq, bq, Wk, bk, Wv, bv, edge_index):
    n, f = feature.shape
    d = W.shape[1]
    e = adj_values.shape[0]

    cpw = -(-e // (NW * CHUNK))
    cpw = -(-cpw // 8) * 8
    e_pad = NW * CHUNK * cpw
    pad = e_pad - e
    n_pad = -(-n // (NS * CHUNK)) * NS * CHUNK
    k = jnp.arange(pad, dtype=jnp.int32)
    row = jnp.concatenate([edge_index[0], n + k % (n_pad - n)]).reshape(-1, CHUNK)
    col = jnp.concatenate([edge_index[1], k % n]).reshape(-1, CHUNK)
    val = jnp.pad(adj_values, (0, pad))
    valx = jnp.broadcast_to(val[:, None], (e_pad, LANES)).reshape(-1)
    zrow = jnp.zeros((CHUNK, d), jnp.float32)

    support = pl.pallas_call(
        _mm_relu_body,
        out_shape=jax.ShapeDtypeStruct((n, d), jnp.float32),
    )(feature, W)

    hop = _make_hop(n_pad, d, cpw)
    p = hop(support, row, col, valx, zrow)
    agg1 = pl.pallas_call(
        _merge_body,
        out_shape=jax.ShapeDtypeStruct((n, d), jnp.float32),
    )(p)
    q = hop(agg1, row, col, valx, zrow)
    out = pl.pallas_call(
        _final_body,
        out_shape=jax.ShapeDtypeStruct((n, d), jnp.float32),
    )(agg1, q, Wv, bv.reshape(1, d))
    return out


# speedup vs baseline: 6.4468x; 1.0315x over previous
import jax
import jax.numpy as jnp
from jax import lax
from jax.experimental import pallas as pl
from jax.experimental.pallas import tpu as pltpu
from jax.experimental.pallas import tpu_sc as plsc

NC = 2
NS = 16
NW = NC * NS
CHUNK = 128
LANES = 16


def _mm_relu_body(f_ref, w_ref, o_ref):
    o_ref[...] = jnp.maximum(
        jnp.dot(f_ref[...], w_ref[...], preferred_element_type=jnp.float32), 0.0
    )


def _merge_body(p_ref, o_ref):
    n = o_ref.shape[0]
    o_ref[...] = p_ref[0, :n, :] + p_ref[1, :n, :]


def _final_body(a1_ref, q_ref, wv_ref, bv_ref, o_ref):
    n = o_ref.shape[0]
    s = a1_ref[...] + q_ref[0, :n, :] + q_ref[1, :n, :]
    o_ref[...] = (
        lax.dot_general(
            s, wv_ref[...], (((1,), (1,)), ((), ())),
            preferred_element_type=jnp.float32,
        )
        + 2.0 * bv_ref[...]
    )


_GDN = lax.GatherDimensionNumbers(
    offset_dims=(), collapsed_slice_dims=(0,), start_index_map=(0,)
)


def _splat(vec16, lane):
    idx = jnp.broadcast_to(lane, (LANES,)).astype(jnp.int32)
    return lax.gather(
        vec16, idx[:, None], _GDN, (1,),
        mode=lax.GatherScatterMode.PROMISE_IN_BOUNDS,
    )


def _make_hop(n_pad, d, cpw):
    rows_per_tile = n_pad // NS
    NB = 2
    NI = 3

    def hop_body(x_hbm, row_hbm, col_hbm, val_hbm, zrow_hbm, out_hbm,
                 row_v, col_v, val_v, rows_v, acc_sh, gsem, isem, ssem):
        c = lax.axis_index("c")
        s = lax.axis_index("s")
        wid = c * NS + s

        pltpu.sync_copy(zrow_hbm, rows_v.at[0])
        for k in range(rows_per_tile // CHUNK):
            pltpu.sync_copy(
                rows_v.at[0],
                acc_sh.at[pl.ds(s * rows_per_tile + k * CHUNK, CHUNK)],
            )
        plsc.subcore_barrier()

        def start_idx(j, b):
            pltpu.async_copy(row_hbm.at[wid * cpw + j], row_v.at[b], isem.at[b])
            pltpu.async_copy(col_hbm.at[wid * cpw + j], col_v.at[b], isem.at[b])
            pltpu.async_copy(val_hbm.at[wid * cpw + j], val_v.at[b], isem.at[b])

        def wait_idx(b):
            pltpu.make_async_copy(row_hbm.at[0], row_v.at[b], isem.at[b]).wait()
            pltpu.make_async_copy(col_hbm.at[0], col_v.at[b], isem.at[b]).wait()
            pltpu.make_async_copy(val_hbm.at[0], val_v.at[b], isem.at[b]).wait()

        start_idx(0, 0)
        wait_idx(0)
        pltpu.async_copy(x_hbm.at[col_v.at[0]], rows_v.at[0], gsem.at[0])
        start_idx(1, 1)

        def chunk_body(j, carry):
            b = lax.rem(j, NB)
            nb = lax.rem(j + 1, NB)
            bi = lax.rem(j, NI)
            nbi = lax.rem(j + 1, NI)

            @pl.when(j >= 1)
            def _():
                pltpu.make_async_copy(
                    rows_v.at[nb], acc_sh.at[row_v.at[0]], ssem.at[nb]
                ).wait()

            @pl.when(j + 1 < cpw)
            def _():
                wait_idx(nbi)
                pltpu.async_copy(
                    x_hbm.at[col_v.at[nbi]], rows_v.at[nb], gsem.at[nb]
                )

            @pl.when(j + 2 < cpw)
            def _():
                start_idx(j + 2, lax.rem(j + 2, NI))

            pltpu.make_async_copy(
                x_hbm.at[pl.ds(0, CHUNK)], rows_v.at[b], gsem.at[b]
            ).wait()

            @plsc.parallel_loop(0, CHUNK, step=LANES)
            def _(g):
                vgrp = val_v[bi, pl.ds(g, LANES)]
                for l in range(LANES):
                    v16 = _splat(vgrp, l)
                    for qq in range(d // LANES):
                        sl = pl.ds(qq * LANES, LANES)
                        rows_v[b, g + l, sl] = rows_v[b, g + l, sl] * v16

            pltpu.async_copy(
                rows_v.at[b], acc_sh.at[row_v.at[bi]], ssem.at[b], add=True
            )
            return carry

        lax.fori_loop(0, cpw, chunk_body, 0)
        lb = lax.rem(jnp.int32(cpw - 1), NB)
        pltpu.make_async_copy(
            rows_v.at[lb], acc_sh.at[row_v.at[0]], ssem.at[lb]
        ).wait()

        plsc.subcore_barrier()
        pltpu.sync_copy(
            acc_sh.at[pl.ds(s * rows_per_tile, rows_per_tile)],
            out_hbm.at[c].at[pl.ds(s * rows_per_tile, rows_per_tile)],
        )

    mesh = plsc.VectorSubcoreMesh(core_axis_name="c", subcore_axis_name="s")
    return pl.kernel(
        hop_body,
        out_type=jax.ShapeDtypeStruct((NC, n_pad, d), jnp.float32),
        mesh=mesh,
        scratch_types=[
            pltpu.VMEM((NI, CHUNK), jnp.int32),
            pltpu.VMEM((NI, CHUNK), jnp.int32),
            pltpu.VMEM((NI, CHUNK), jnp.float32),
            pltpu.VMEM((NB, CHUNK, d), jnp.float32),
            pltpu.VMEM_SHARED((n_pad, d), jnp.float32),
            pltpu.SemaphoreType.DMA((NB,)),
            pltpu.SemaphoreType.DMA((NI,)),
            pltpu.SemaphoreType.DMA((NB,)),
        ],
    )


@jax.jit
def kernel(feature, adj_values, W, Wq, bq, Wk, bk, Wv, bv, edge_index):
    n, f = feature.shape
    d = W.shape[1]
    e = adj_values.shape[0]

    cpw = -(-e // (NW * CHUNK))
    cpw = -(-cpw // 8) * 8
    e_pad = NW * CHUNK * cpw
    pad = e_pad - e
    n_pad = -(-n // (NS * CHUNK)) * NS * CHUNK
    k = jnp.arange(pad, dtype=jnp.int32)
    row = jnp.concatenate([edge_index[0], n + k % (n_pad - n)]).reshape(-1, CHUNK)
    col = jnp.concatenate([edge_index[1], k % n]).reshape(-1, CHUNK)
    val = jnp.pad(adj_values, (0, pad)).reshape(-1, CHUNK)
    zrow = jnp.zeros((CHUNK, d), jnp.float32)

    support = pl.pallas_call(
        _mm_relu_body,
        out_shape=jax.ShapeDtypeStruct((n, d), jnp.float32),
    )(feature, W)

    hop = _make_hop(n_pad, d, cpw)
    p = hop(support, row, col, val, zrow)
    agg1 = pl.pallas_call(
        _merge_body,
        out_shape=jax.ShapeDtypeStruct((n, d), jnp.float32),
    )(p)
    q = hop(agg1, row, col, val, zrow)
    out = pl.pallas_call(
        _final_body,
        out_shape=jax.ShapeDtypeStruct((n, d), jnp.float32),
    )(agg1, q, Wv, bv.reshape(1, d))
    return out
